# Initial kernel scaffold; baseline (speedup 1.0000x reference)
import jax
import jax.numpy as jnp
from jax.experimental import pallas as pl


def kernel(x):
    raise NotImplementedError("write your pallas kernel here")


# speedup vs baseline: 7.1749x; 7.1749x over previous
import functools

import jax
import jax.numpy as jnp
from jax import lax
from jax.experimental import pallas as pl
from jax.experimental.pallas import tpu as pltpu
from jax.experimental.pallas import tpu_sc as plsc

B = 4
H = 384
PS = 8
W = H - PS + 1
K = 32
CAP = 48
NSUB = 8
ROWS = 48
MAXI = 2147483647


def _iota():
    return lax.iota(jnp.int32, 16)


def _scalar(v):
    return jnp.max(v)


def _popcount(mask):
    return jnp.max(plsc.all_reduce_population_count(mask))


def _find_bucket(hist, compact, k):
    iot = _iota()

    def chunk(i, carry):
        bst, tot = carry
        j = 15 - i
        acc = hist[0, pl.ds(16 * j, 16)]
        for l in range(1, 16):
            acc = acc + hist[l, pl.ds(16 * j, 16)]
        compact[pl.ds(16 * j, 16)] = acc
        suf = lax.rev(plsc.cumsum(lax.rev(acc, (0,))), (0,)) + tot
        cnt = _popcount(suf >= k)
        bc = jnp.where(cnt > 0, 16 * j + cnt - 1, -1)
        return jnp.maximum(bst, bc), tot + jnp.sum(acc)

    bucket, _ = lax.fori_loop(0, 16, chunk, (jnp.int32(-1), jnp.int32(0)))

    def above(t, accv):
        v = compact[pl.ds(16 * t, 16)]
        return accv + jnp.where(16 * t + iot > bucket, v, 0)

    accv = lax.fori_loop(0, 16, above, jnp.zeros((16,), jnp.int32))
    return bucket, jnp.sum(accv)


def _clear_hist(hist):
    z = jnp.zeros((16,), jnp.int32)

    def body(l, _):
        for j in range(16):
            hist[l, pl.ds(16 * j, 16)] = z
        return 0

    lax.fori_loop(0, 16, body, 0)


def _sc_patchify(x):
    mesh = plsc.VectorSubcoreMesh(
        core_axis_name="c", subcore_axis_name="s", num_cores=2,
        num_subcores=16)

    @functools.partial(
        pl.kernel,
        out_type=jax.ShapeDtypeStruct((B, K, 64), jnp.float32),
        mesh=mesh,
        compiler_params=pltpu.CompilerParams(use_tc_tiling_on_sc=False,
                                             needs_layout_passes=False),
        scratch_types=[
            pltpu.VMEM((ROWS + 8, H), jnp.float32),
            pltpu.VMEM((ROWS * H,), jnp.int32),
            pltpu.VMEM((16, 256), jnp.int32),
            pltpu.VMEM((256,), jnp.int32),
            pltpu.VMEM((CAP,), jnp.int32),
            pltpu.VMEM((CAP,), jnp.int32),
            pltpu.VMEM((NSUB * CAP,), jnp.int32),
            pltpu.VMEM((NSUB * CAP,), jnp.int32),
            pltpu.VMEM((K,), jnp.int32),
            pltpu.VMEM((80,), jnp.float32),
            pltpu.VMEM_SHARED((2, NSUB * CAP), jnp.int32),
            pltpu.VMEM_SHARED((2, NSUB * CAP), jnp.int32),
            pltpu.VMEM_SHARED((2, K), jnp.int32),
        ],
    )
    def k(x_hbm, out_hbm, buf, keys, hist, compact, kloc, iloc, mkeys, midx,
          winv, pstg, keys_s, idx_s, win_s):
        cid = lax.axis_index("c")
        sid = lax.axis_index("s")
        bslot = sid // NSUB
        b = 2 * cid + bslot
        o = sid % NSUB
        r0 = ROWS * o
        n_rows = jnp.where(o == NSUB - 1, W - ROWS * (NSUB - 1), ROWS)
        iot = _iota()
        ones = jnp.full((16,), 1, jnp.int32)

        @pl.when(o < NSUB - 1)
        def _():
            pltpu.sync_copy(x_hbm.at[b, pl.ds(r0, ROWS + 8)], buf)

        @pl.when(o == NSUB - 1)
        def _():
            pltpu.sync_copy(x_hbm.at[b, pl.ds(ROWS * (NSUB - 1), H - ROWS *
                                              (NSUB - 1))],
                            buf.at[pl.ds(0, H - ROWS * (NSUB - 1))])

        _clear_hist(hist)

        def r1(r, _):
            for c in range(24):
                if c < 23:
                    px = buf[r + 4, pl.ds(16 * c + 4, 16)]
                    key = lax.bitcast_convert_type(px, jnp.int32)
                else:
                    colpix = jnp.minimum(368 + iot, W - 1) + 4
                    px = plsc.load_gather(buf, [jnp.full((16,), r + 4,
                                                         jnp.int32), colpix])
                    key = lax.bitcast_convert_type(px, jnp.int32)
                    key = jnp.where(368 + iot < W, key, -1)
                keys[pl.ds(r * H + 16 * c, 16)] = key
                plsc.addupdate_scatter(hist, [iot, key >> 23], ones,
                                       mask=key >= 0)
            return 0

        lax.fori_loop(0, n_rows, r1, 0)
        b1, ca1 = _find_bucket(hist, compact, K)
        k2 = K - ca1

        _clear_hist(hist)
        n_chunks = n_rows * (H // 16)

        def r2(t, _):
            key = keys[pl.ds(16 * t, 16)]
            plsc.addupdate_scatter(hist, [iot, (key >> 15) & 255], ones,
                                   mask=(key >> 23) == b1)
            return 0

        lax.fori_loop(0, n_chunks, r2, 0)
        b2, ca2 = _find_bucket(hist, compact, k2)
        k3 = k2 - ca2

        _clear_hist(hist)
        pref16 = (b1 << 8) | b2

        def r3(t, _):
            key = keys[pl.ds(16 * t, 16)]
            plsc.addupdate_scatter(hist, [iot, (key >> 7) & 255], ones,
                                   mask=(key >> 15) == pref16)
            return 0

        lax.fori_loop(0, n_chunks, r3, 0)
        b3, _ca3 = _find_bucket(hist, compact, k3)
        thresh = (b1 << 23) | (b2 << 15) | (b3 << 7)

        neg1 = jnp.full((16,), -1, jnp.int32)
        for t in range(CAP // 16):
            kloc[pl.ds(16 * t, 16)] = neg1
            iloc[pl.ds(16 * t, 16)] = neg1

        def collect(t, off):
            key = keys[pl.ds(16 * t, 16)]
            m = key >= thresh
            f = 16 * t + iot
            idx = (r0 + f // H) * W + (f - (f // H) * H)
            off16 = jnp.minimum(off, CAP - 16)
            plsc.store_compressed(kloc.at[pl.ds(off16, 16)], key, mask=m)
            plsc.store_compressed(iloc.at[pl.ds(off16, 16)], idx, mask=m)
            return off + _popcount(m)

        lax.fori_loop(0, n_chunks, collect, jnp.int32(0))
        pltpu.sync_copy(kloc, keys_s.at[bslot, pl.ds(CAP * o, CAP)])
        pltpu.sync_copy(iloc, idx_s.at[bslot, pl.ds(CAP * o, CAP)])
        plsc.subcore_barrier()

        @pl.when(o == 0)
        def _():
            pltpu.sync_copy(keys_s.at[bslot], mkeys)
            pltpu.sync_copy(idx_s.at[bslot], midx)
            nch = NSUB * CAP // 16

            for rank in range(K):
                def pmax(t, acc):
                    return jnp.maximum(acc, mkeys[pl.ds(16 * t, 16)])

                m = jnp.max(lax.fori_loop(0, nch, pmax, neg1))

                def pimin(t, acc):
                    kv = mkeys[pl.ds(16 * t, 16)]
                    iv = midx[pl.ds(16 * t, 16)]
                    return jnp.minimum(acc, jnp.where(kv == m, iv, MAXI))

                imin = jnp.min(lax.fori_loop(0, nch, pimin, jnp.full(
                    (16,), MAXI, jnp.int32)))

                def pslot(t, acc):
                    kv = mkeys[pl.ds(16 * t, 16)]
                    iv = midx[pl.ds(16 * t, 16)]
                    hit = (kv == m) & (iv == imin)
                    return jnp.minimum(acc, jnp.where(hit, 16 * t + iot,
                                                      MAXI))

                p = jnp.min(lax.fori_loop(0, nch, pslot, jnp.full(
                    (16,), MAXI, jnp.int32)))
                plsc.store_scatter(
                    winv, [jnp.full((16,), rank, jnp.int32)],
                    jnp.full((16,), imin, jnp.int32), mask=iot == 0)
                plsc.store_scatter(mkeys, [jnp.full((16,), p, jnp.int32)],
                                   neg1, mask=iot == 0)

            pltpu.sync_copy(winv, win_s.at[bslot])

        plsc.subcore_barrier()

        pltpu.sync_copy(win_s.at[bslot], winv)
        wvecs = [winv[pl.ds(16 * t, 16)] for t in range(K // 16)]
        for j in range(K):
            w = jnp.max(jnp.where(iot == j % 16, wvecs[j // 16], -1))
            r = w // W
            c = w - r * W
            rl = r - r0

            @pl.when((rl >= 0) & (rl < n_rows))
            def _():
                colidx = jnp.minimum(c + iot, H - 1)
                for t in range(PS):
                    row = plsc.load_gather(
                        buf, [jnp.full((16,), rl + t, jnp.int32), colidx])
                    plsc.store_compressed(pstg.at[pl.ds(PS * t, 16)], row,
                                          mask=iot < PS)
                pltpu.sync_copy(pstg.at[pl.ds(0, 64)], out_hbm.at[b, j])

    return k(x)


def kernel(x):
    xb, c, h, w = x.shape
    assert (xb, c, h, w) == (B, 1, H, H)
    out = _sc_patchify(x.reshape(B, H, H))
    return out.reshape(B, K, 1, PS, PS)


# speedup vs baseline: 8.9939x; 1.2535x over previous
import functools

import jax
import jax.numpy as jnp
from jax import lax
from jax.experimental import pallas as pl
from jax.experimental.pallas import tpu as pltpu
from jax.experimental.pallas import tpu_sc as plsc

B = 4
H = 384
PS = 8
W = H - PS + 1
K = 32
CAP = 48
NSUB = 8
ROWS = 48
MAXI = 2147483647


def _iota():
    return lax.iota(jnp.int32, 16)


def _scalar(v):
    return jnp.max(v)


def _popcount(mask):
    return jnp.max(plsc.all_reduce_population_count(mask))


def _find_bucket(hist, compact, k):
    iot = _iota()

    def chunk(i, carry):
        bst, tot = carry
        j = 15 - i
        acc = hist[0, pl.ds(16 * j, 16)]
        for l in range(1, 16):
            acc = acc + hist[l, pl.ds(16 * j, 16)]
        compact[pl.ds(16 * j, 16)] = acc
        suf = lax.rev(plsc.cumsum(lax.rev(acc, (0,))), (0,)) + tot
        cnt = _popcount(suf >= k)
        bc = jnp.where(cnt > 0, 16 * j + cnt - 1, -1)
        return jnp.maximum(bst, bc), tot + jnp.sum(acc)

    bucket, _ = lax.fori_loop(0, 16, chunk, (jnp.int32(-1), jnp.int32(0)))

    def above(t, accv):
        v = compact[pl.ds(16 * t, 16)]
        return accv + jnp.where(16 * t + iot > bucket, v, 0)

    accv = lax.fori_loop(0, 16, above, jnp.zeros((16,), jnp.int32))
    return bucket, jnp.sum(accv)


def _clear_hist(hist):
    z = jnp.zeros((16,), jnp.int32)

    def body(l, _):
        for j in range(16):
            hist[l, pl.ds(16 * j, 16)] = z
        return 0

    lax.fori_loop(0, 16, body, 0)


def _sc_patchify(x):
    mesh = plsc.VectorSubcoreMesh(
        core_axis_name="c", subcore_axis_name="s", num_cores=2,
        num_subcores=16)

    @functools.partial(
        pl.kernel,
        out_type=jax.ShapeDtypeStruct((B, K, 64), jnp.float32),
        mesh=mesh,
        compiler_params=pltpu.CompilerParams(use_tc_tiling_on_sc=False,
                                             needs_layout_passes=False),
        scratch_types=[
            pltpu.VMEM((ROWS + 8, H), jnp.float32),
            pltpu.VMEM((ROWS * H,), jnp.int32),
            pltpu.VMEM((16, 256), jnp.int32),
            pltpu.VMEM((256,), jnp.int32),
            pltpu.VMEM((CAP,), jnp.int32),
            pltpu.VMEM((CAP,), jnp.int32),
            pltpu.VMEM((NSUB * CAP,), jnp.int32),
            pltpu.VMEM((NSUB * CAP,), jnp.int32),
            pltpu.VMEM((K,), jnp.int32),
            pltpu.VMEM((80,), jnp.float32),
            pltpu.VMEM_SHARED((2, NSUB * CAP), jnp.int32),
            pltpu.VMEM_SHARED((2, NSUB * CAP), jnp.int32),
            pltpu.VMEM_SHARED((2, K), jnp.int32),
        ],
    )
    def k(x_hbm, out_hbm, buf, keys, hist, compact, kloc, iloc, mkeys, midx,
          winv, pstg, keys_s, idx_s, win_s):
        cid = lax.axis_index("c")
        sid = lax.axis_index("s")
        bslot = sid // NSUB
        b = 2 * cid + bslot
        o = sid % NSUB
        r0 = ROWS * o
        n_rows = jnp.where(o == NSUB - 1, W - ROWS * (NSUB - 1), ROWS)
        iot = _iota()
        ones = jnp.full((16,), 1, jnp.int32)

        @pl.when(o < NSUB - 1)
        def _():
            pltpu.sync_copy(x_hbm.at[b, pl.ds(r0, ROWS + 8)], buf)

        @pl.when(o == NSUB - 1)
        def _():
            pltpu.sync_copy(x_hbm.at[b, pl.ds(ROWS * (NSUB - 1), H - ROWS *
                                              (NSUB - 1))],
                            buf.at[pl.ds(0, H - ROWS * (NSUB - 1))])

        _clear_hist(hist)

        def r1(r, _):
            for c in range(24):
                if c < 23:
                    px = buf[r + 4, pl.ds(16 * c + 4, 16)]
                    key = lax.bitcast_convert_type(px, jnp.int32)
                else:
                    colpix = jnp.minimum(368 + iot, W - 1) + 4
                    px = plsc.load_gather(buf, [jnp.full((16,), r + 4,
                                                         jnp.int32), colpix])
                    key = lax.bitcast_convert_type(px, jnp.int32)
                    key = jnp.where(368 + iot < W, key, -1)
                keys[pl.ds(r * H + 16 * c, 16)] = key
                plsc.addupdate_scatter(hist, [iot, key >> 23], ones,
                                       mask=key >= 0)
            return 0

        lax.fori_loop(0, n_rows, r1, 0)
        b1, ca1 = _find_bucket(hist, compact, K)
        k2 = K - ca1

        _clear_hist(hist)
        n_chunks = n_rows * (H // 16)

        def r2(i, _):
            for u in range(8):
                key = keys[pl.ds(128 * i + 16 * u, 16)]
                plsc.addupdate_scatter(hist, [iot, (key >> 15) & 255], ones,
                                       mask=(key >> 23) == b1)
            return 0

        lax.fori_loop(0, n_chunks // 8, r2, 0)
        b2, ca2 = _find_bucket(hist, compact, k2)
        k3 = k2 - ca2

        _clear_hist(hist)
        pref16 = (b1 << 8) | b2

        def r3(i, _):
            for u in range(8):
                key = keys[pl.ds(128 * i + 16 * u, 16)]
                plsc.addupdate_scatter(hist, [iot, (key >> 7) & 255], ones,
                                       mask=(key >> 15) == pref16)
            return 0

        lax.fori_loop(0, n_chunks // 8, r3, 0)
        b3, _ca3 = _find_bucket(hist, compact, k3)
        thresh = (b1 << 23) | (b2 << 15) | (b3 << 7)

        neg1 = jnp.full((16,), -1, jnp.int32)
        for t in range(CAP // 16):
            kloc[pl.ds(16 * t, 16)] = neg1
            iloc[pl.ds(16 * t, 16)] = neg1

        def collect(r, offv):
            rowb = (r0 + r) * W
            for c in range(24):
                key = keys[pl.ds(r * H + 16 * c, 16)]
                m = key >= thresh
                mi = jnp.where(m, 1, 0)
                pos = jnp.minimum(offv + plsc.cumsum(mi) - 1, CAP - 1)
                plsc.store_scatter(kloc, [pos], key, mask=m)
                plsc.store_scatter(iloc, [pos], rowb + 16 * c + iot, mask=m)
                offv = offv + plsc.all_reduce_population_count(m)
            return offv

        lax.fori_loop(0, n_rows, collect, jnp.zeros((16,), jnp.int32))
        pltpu.sync_copy(kloc, keys_s.at[bslot, pl.ds(CAP * o, CAP)])
        pltpu.sync_copy(iloc, idx_s.at[bslot, pl.ds(CAP * o, CAP)])
        plsc.subcore_barrier()

        @pl.when(o == 0)
        def _():
            pltpu.sync_copy(keys_s.at[bslot], mkeys)
            pltpu.sync_copy(idx_s.at[bslot], midx)
            nch = NSUB * CAP // 16

            mprev = None
            for rank in range(K):
                if mprev is None:
                    def pmax(t, acc):
                        return jnp.maximum(acc, mkeys[pl.ds(16 * t, 16)])
                else:
                    mp, ip = mprev

                    def pmax(t, acc):
                        kv = mkeys[pl.ds(16 * t, 16)]
                        iv = midx[pl.ds(16 * t, 16)]
                        kv = jnp.where((kv == mp) & (iv == ip), -1, kv)
                        mkeys[pl.ds(16 * t, 16)] = kv
                        return jnp.maximum(acc, kv)

                m = jnp.max(lax.fori_loop(0, nch, pmax, neg1))

                def pimin(t, acc):
                    kv = mkeys[pl.ds(16 * t, 16)]
                    iv = midx[pl.ds(16 * t, 16)]
                    return jnp.minimum(acc, jnp.where(kv == m, iv, MAXI))

                imin = jnp.min(lax.fori_loop(0, nch, pimin, jnp.full(
                    (16,), MAXI, jnp.int32)))
                plsc.store_scatter(
                    winv, [jnp.full((16,), rank, jnp.int32)],
                    jnp.full((16,), imin, jnp.int32), mask=iot == 0)
                mprev = (m, imin)

            pltpu.sync_copy(winv, win_s.at[bslot])

        plsc.subcore_barrier()

        pltpu.sync_copy(win_s.at[bslot], winv)
        wvecs = [winv[pl.ds(16 * t, 16)] for t in range(K // 16)]
        for j in range(K):
            w = jnp.max(jnp.where(iot == j % 16, wvecs[j // 16], -1))
            r = w // W
            c = w - r * W
            rl = r - r0

            @pl.when((rl >= 0) & (rl < n_rows))
            def _():
                colidx = jnp.minimum(c + iot, H - 1)
                for t in range(PS):
                    row = plsc.load_gather(
                        buf, [jnp.full((16,), rl + t, jnp.int32), colidx])
                    plsc.store_compressed(pstg.at[pl.ds(PS * t, 16)], row,
                                          mask=iot < PS)
                pltpu.sync_copy(pstg.at[pl.ds(0, 64)], out_hbm.at[b, j])

    return k(x)


def kernel(x):
    xb, c, h, w = x.shape
    assert (xb, c, h, w) == (B, 1, H, H)
    out = _sc_patchify(x.reshape(B, H, H))
    return out.reshape(B, K, 1, PS, PS)


# speedup vs baseline: 9.8094x; 1.0907x over previous
import functools

import jax
import jax.numpy as jnp
from jax import lax
from jax.experimental import pallas as pl
from jax.experimental.pallas import tpu as pltpu
from jax.experimental.pallas import tpu_sc as plsc

B = 4
H = 384
PS = 8
W = H - PS + 1
K = 32
CAP = 48
NSUB = 8
ROWS = 48
MAXI = 2147483647


def _iota():
    return lax.iota(jnp.int32, 16)


def _scalar(v):
    return jnp.max(v)


def _popcount(mask):
    return jnp.max(plsc.all_reduce_population_count(mask))


def _find_bucket(hist, compact, k):
    iot = _iota()

    def chunk(i, carry):
        bst, tot = carry
        j = 15 - i
        acc = hist[0, pl.ds(16 * j, 16)]
        for l in range(1, 16):
            acc = acc + hist[l, pl.ds(16 * j, 16)]
        compact[pl.ds(16 * j, 16)] = acc
        suf = lax.rev(plsc.cumsum(lax.rev(acc, (0,))), (0,)) + tot
        cnt = _popcount(suf >= k)
        bc = jnp.where(cnt > 0, 16 * j + cnt - 1, -1)
        return jnp.maximum(bst, bc), tot + jnp.sum(acc)

    bucket, _ = lax.fori_loop(0, 16, chunk, (jnp.int32(-1), jnp.int32(0)))

    def above(t, accv):
        v = compact[pl.ds(16 * t, 16)]
        return accv + jnp.where(16 * t + iot > bucket, v, 0)

    accv = lax.fori_loop(0, 16, above, jnp.zeros((16,), jnp.int32))
    return bucket, jnp.sum(accv)


def _clear_hist(hist):
    z = jnp.zeros((16,), jnp.int32)

    def body(l, _):
        for j in range(16):
            hist[l, pl.ds(16 * j, 16)] = z
        return 0

    lax.fori_loop(0, 16, body, 0)


def _sc_patchify(x):
    mesh = plsc.VectorSubcoreMesh(
        core_axis_name="c", subcore_axis_name="s", num_cores=2,
        num_subcores=16)

    @functools.partial(
        pl.kernel,
        out_type=jax.ShapeDtypeStruct((B, K, 64), jnp.float32),
        mesh=mesh,
        compiler_params=pltpu.CompilerParams(use_tc_tiling_on_sc=False,
                                             needs_layout_passes=False),
        scratch_types=[
            pltpu.VMEM((ROWS + 8, H), jnp.float32),
            pltpu.VMEM((ROWS * H,), jnp.int32),
            pltpu.VMEM((16, 256), jnp.int32),
            pltpu.VMEM((256,), jnp.int32),
            pltpu.VMEM((CAP,), jnp.int32),
            pltpu.VMEM((CAP,), jnp.int32),
            pltpu.VMEM((CAP,), jnp.int32),
            pltpu.VMEM((CAP,), jnp.int32),
            pltpu.VMEM((NSUB * CAP,), jnp.int32),
            pltpu.VMEM((NSUB * CAP,), jnp.int32),
            pltpu.VMEM((K,), jnp.int32),
            pltpu.VMEM((80,), jnp.float32),
            pltpu.VMEM_SHARED((2, NSUB * CAP), jnp.int32),
            pltpu.VMEM_SHARED((2, NSUB * CAP), jnp.int32),
            pltpu.VMEM_SHARED((2, K), jnp.int32),
        ],
    )
    def k(x_hbm, out_hbm, buf, keys, hist, compact, kloc, iloc, ksort, isort,
          mkeys, midx, winv, pstg, keys_s, idx_s, win_s):
        cid = lax.axis_index("c")
        sid = lax.axis_index("s")
        bslot = sid // NSUB
        b = 2 * cid + bslot
        o = sid % NSUB
        r0 = ROWS * o
        n_rows = jnp.where(o == NSUB - 1, W - ROWS * (NSUB - 1), ROWS)
        iot = _iota()
        ones = jnp.full((16,), 1, jnp.int32)

        @pl.when(o < NSUB - 1)
        def _():
            pltpu.sync_copy(x_hbm.at[b, pl.ds(r0, ROWS + 8)], buf)

        @pl.when(o == NSUB - 1)
        def _():
            pltpu.sync_copy(x_hbm.at[b, pl.ds(ROWS * (NSUB - 1), H - ROWS *
                                              (NSUB - 1))],
                            buf.at[pl.ds(0, H - ROWS * (NSUB - 1))])

        _clear_hist(hist)

        def r1(r, _):
            for c in range(24):
                if c < 23:
                    px = buf[r + 4, pl.ds(16 * c + 4, 16)]
                    key = lax.bitcast_convert_type(px, jnp.int32)
                else:
                    colpix = jnp.minimum(368 + iot, W - 1) + 4
                    px = plsc.load_gather(buf, [jnp.full((16,), r + 4,
                                                         jnp.int32), colpix])
                    key = lax.bitcast_convert_type(px, jnp.int32)
                    key = jnp.where(368 + iot < W, key, -1)
                keys[pl.ds(r * H + 16 * c, 16)] = key
                plsc.addupdate_scatter(hist, [iot, key >> 23], ones,
                                       mask=key >= 0)
            return 0

        lax.fori_loop(0, n_rows, r1, 0)
        b1, ca1 = _find_bucket(hist, compact, K)
        k2 = K - ca1

        _clear_hist(hist)
        n_chunks = n_rows * (H // 16)

        def r2(i, _):
            for u in range(8):
                key = keys[pl.ds(128 * i + 16 * u, 16)]
                plsc.addupdate_scatter(hist, [iot, (key >> 15) & 255], ones,
                                       mask=(key >> 23) == b1)
            return 0

        lax.fori_loop(0, n_chunks // 8, r2, 0)
        b2, ca2 = _find_bucket(hist, compact, k2)
        k3 = k2 - ca2

        _clear_hist(hist)
        pref16 = (b1 << 8) | b2

        def r3(i, _):
            for u in range(8):
                key = keys[pl.ds(128 * i + 16 * u, 16)]
                plsc.addupdate_scatter(hist, [iot, (key >> 7) & 255], ones,
                                       mask=(key >> 15) == pref16)
            return 0

        lax.fori_loop(0, n_chunks // 8, r3, 0)
        b3, _ca3 = _find_bucket(hist, compact, k3)
        thresh = (b1 << 23) | (b2 << 15) | (b3 << 7)

        neg1 = jnp.full((16,), -1, jnp.int32)
        for t in range(CAP // 16):
            kloc[pl.ds(16 * t, 16)] = neg1
            iloc[pl.ds(16 * t, 16)] = neg1

        def collect(r, offv):
            rowb = (r0 + r) * W
            for c in range(24):
                key = keys[pl.ds(r * H + 16 * c, 16)]
                m = key >= thresh
                mi = jnp.where(m, 1, 0)
                pos = jnp.minimum(offv + plsc.cumsum(mi) - 1, CAP - 1)
                plsc.store_scatter(kloc, [pos], key, mask=m)
                plsc.store_scatter(iloc, [pos], rowb + 16 * c + iot, mask=m)
                offv = offv + plsc.all_reduce_population_count(m)
            return offv

        lax.fori_loop(0, n_rows, collect, jnp.zeros((16,), jnp.int32))

        for t in range(CAP // 16):
            ksort[pl.ds(16 * t, 16)] = neg1
            isort[pl.ds(16 * t, 16)] = neg1

        def lsort(rank, carry):
            mp, ip = carry
            macc = neg1
            for t in range(CAP // 16):
                kv = kloc[pl.ds(16 * t, 16)]
                iv = iloc[pl.ds(16 * t, 16)]
                kv = jnp.where((kv == mp) & (iv == ip), -1, kv)
                kloc[pl.ds(16 * t, 16)] = kv
                macc = jnp.maximum(macc, kv)
            m = jnp.max(macc)
            iacc = jnp.full((16,), MAXI, jnp.int32)
            for t in range(CAP // 16):
                kv = kloc[pl.ds(16 * t, 16)]
                iv = iloc[pl.ds(16 * t, 16)]
                iacc = jnp.minimum(iacc, jnp.where(kv == m, iv, MAXI))
            im = jnp.min(iacc)
            rk = jnp.full((16,), rank, jnp.int32)
            plsc.store_scatter(ksort, [rk], jnp.full((16,), m, jnp.int32),
                               mask=iot == 0)
            plsc.store_scatter(isort, [rk], jnp.full((16,), im, jnp.int32),
                               mask=iot == 0)
            return m, im

        lax.fori_loop(0, K, lsort, (jnp.int32(-2), jnp.int32(-2)))
        pltpu.sync_copy(ksort, keys_s.at[bslot, pl.ds(CAP * o, CAP)])
        pltpu.sync_copy(isort, idx_s.at[bslot, pl.ds(CAP * o, CAP)])
        plsc.subcore_barrier()

        @pl.when(o == 0)
        def _():
            pltpu.sync_copy(keys_s.at[bslot], mkeys)
            pltpu.sync_copy(idx_s.at[bslot], midx)
            lane8 = jnp.minimum(iot, NSUB - 1)
            valid8 = iot < NSUB

            def step(rank, ptrv):
                addr = lane8 * CAP + ptrv
                kh = plsc.load_gather(mkeys, [addr])
                ih = plsc.load_gather(midx, [addr])
                kh = jnp.where(valid8, kh, -1)
                m = jnp.max(kh)
                im = jnp.min(jnp.where(kh == m, ih, MAXI))
                hit = (kh == m) & (ih == im) & valid8
                plsc.store_scatter(
                    winv, [jnp.full((16,), rank, jnp.int32)],
                    jnp.full((16,), im, jnp.int32), mask=iot == 0)
                return ptrv + jnp.where(hit, 1, 0)

            lax.fori_loop(0, K, step, jnp.zeros((16,), jnp.int32))
            pltpu.sync_copy(winv, win_s.at[bslot])

        plsc.subcore_barrier()

        pltpu.sync_copy(win_s.at[bslot], winv)
        wvecs = [winv[pl.ds(16 * t, 16)] for t in range(K // 16)]
        for j in range(K):
            w = jnp.max(jnp.where(iot == j % 16, wvecs[j // 16], -1))
            r = w // W
            c = w - r * W
            rl = r - r0

            @pl.when((rl >= 0) & (rl < n_rows))
            def _():
                colidx = jnp.minimum(c + iot, H - 1)
                for t in range(PS):
                    row = plsc.load_gather(
                        buf, [jnp.full((16,), rl + t, jnp.int32), colidx])
                    plsc.store_compressed(pstg.at[pl.ds(PS * t, 16)], row,
                                          mask=iot < PS)
                pltpu.sync_copy(pstg.at[pl.ds(0, 64)], out_hbm.at[b, j])

    return k(x)


def kernel(x):
    xb, c, h, w = x.shape
    assert (xb, c, h, w) == (B, 1, H, H)
    out = _sc_patchify(x.reshape(B, H, H))
    return out.reshape(B, K, 1, PS, PS)
